# Initial kernel scaffold; baseline (speedup 1.0000x reference)
import jax
import jax.numpy as jnp
from jax.experimental import pallas as pl


def kernel(coords, x_line, y_line, z_line, grid):
    raise NotImplementedError("write your pallas kernel here")


# speedup vs baseline: 1.8561x; 1.8561x over previous
import functools

import jax
import jax.numpy as jnp
from jax import lax
from jax.experimental import pallas as pl
from jax.experimental.pallas import tpu as pltpu
from jax.experimental.pallas import tpu_sc as plsc

NC = 2
NS = 16
L = 16
NW = NC * NS

M = 256
IDX_W = 128
NSTREAM = M // IDX_W

_GATHER_DNUMS = lax.GatherDimensionNumbers(
    offset_dims=(), collapsed_slice_dims=(0,), start_index_map=(0,))


def _bcast_lane(vec, r):
    idx = jnp.full((L, 1), r, jnp.int32)
    return lax.gather(vec, idx, _GATHER_DNUMS, (1,),
                      mode=lax.GatherScatterMode.PROMISE_IN_BOUNDS)


def _make_triline(B, N, C):
    BW = B // NW
    SUB = BW // M

    mesh = plsc.VectorSubcoreMesh(core_axis_name="c", subcore_axis_name="s",
                                  num_cores=NC, num_subcores=NS)

    set_types = (
        [pltpu.VMEM((M,), jnp.float32)] * 3
        + [pltpu.VMEM((IDX_W,), jnp.int32)] * (6 * NSTREAM)
        + [pltpu.VMEM((M,), jnp.float32)] * 3
        + [pltpu.VMEM((M, C), jnp.float32)] * 6
    )
    scratch = ([pltpu.VMEM((16,), jnp.float32)]
               + set_types + set_types
               + [pltpu.SemaphoreType.DMA, pltpu.SemaphoreType.DMA])

    def _split_set(scr):
        coord_refs = scr[0:3]
        idx_refs = [scr[3 + g * NSTREAM: 3 + (g + 1) * NSTREAM]
                    for g in range(6)]
        w_refs = scr[3 + 6 * NSTREAM: 6 + 6 * NSTREAM]
        bufs = scr[6 + 6 * NSTREAM: 12 + 6 * NSTREAM]
        return coord_refs, idx_refs, w_refs, bufs

    SET_LEN = 12 + 6 * NSTREAM

    @functools.partial(
        pl.kernel,
        out_type=jax.ShapeDtypeStruct((B, C), jnp.float32),
        mesh=mesh,
        scratch_types=scratch,
        compiler_params=pltpu.CompilerParams(use_tc_tiling_on_sc=False),
    )
    def triline(xs, ys, zs, xl, yl, zl, grid, out, *scr):
        gw = scr[0]
        set_a = _split_set(scr[1:1 + SET_LEN])
        set_b = _split_set(scr[1 + SET_LEN:1 + 2 * SET_LEN])
        sem_a = scr[1 + 2 * SET_LEN]
        sem_b = scr[2 + 2 * SET_LEN]

        wid = lax.axis_index("s") * NC + lax.axis_index("c")
        base = wid * BW

        pltpu.sync_copy(grid.at[pl.ds(0, 16)], gw)
        g16 = gw[...]
        g0 = _bcast_lane(g16, 0)
        inv_dx = 1.0 / (_bcast_lane(g16, 1) - g0)

        tables = (xl, xl, yl, yl, zl, zl)

        def prep(bset, sem, s):
            coord_refs, idx_refs, w_refs, bufs = bset
            off = base + s * M
            for c_ref, src in zip(coord_refs, (xs, ys, zs)):
                pltpu.sync_copy(src.at[pl.ds(off, M)], c_ref)
            for i in range(M // L):
                sl = pl.ds(i * L, L)
                j, k = divmod(i, IDX_W // L)
                jsl = pl.ds(k * L, L)
                for a in range(3):
                    v = coord_refs[a][sl]
                    pos = (v - g0) * inv_dx
                    idx0 = jnp.clip(pos.astype(jnp.int32), 0, N - 2)
                    idx_refs[2 * a][j][jsl] = idx0
                    idx_refs[2 * a + 1][j][jsl] = idx0 + 1
                    w_refs[a][sl] = pos - idx0.astype(jnp.float32)
            for g in range(6):
                for j in range(NSTREAM):
                    pltpu.async_copy(tables[g].at[idx_refs[g][j]],
                                     bufs[g].at[pl.ds(j * IDX_W, IDX_W)], sem)

        def finish(bset, sem, s):
            coord_refs, idx_refs, w_refs, bufs = bset
            off = base + s * M
            for g in range(6):
                for j in range(NSTREAM):
                    pltpu.make_async_copy(
                        tables[g].at[idx_refs[g][j]],
                        bufs[g].at[pl.ds(j * IDX_W, IDX_W)], sem).wait()
            bx0, bx1, by0, by1, bz0, bz1 = bufs

            def combine(gr, carry2):
                r0 = gr * L
                wsl = pl.ds(r0, L)
                wx16 = w_refs[0][wsl]
                wy16 = w_refs[1][wsl]
                wz16 = w_refs[2][wsl]
                for r in range(L):
                    row = r0 + r
                    wxv = _bcast_lane(wx16, r)
                    wyv = _bcast_lane(wy16, r)
                    wzv = _bcast_lane(wz16, r)
                    for h in range(C // L):
                        csl = pl.ds(h * L, L)
                        fx0 = bx0[row, csl]
                        fx1 = bx1[row, csl]
                        fy0 = by0[row, csl]
                        fy1 = by1[row, csl]
                        fz0 = bz0[row, csl]
                        fz1 = bz1[row, csl]
                        acc = (fx0 * (1.0 - wxv) + fx1 * wxv
                               + fy0 * (1.0 - wyv) + fy1 * wyv
                               + fz0 * (1.0 - wzv) + fz1 * wzv)
                        bx0[row, csl] = acc
                return carry2

            lax.fori_loop(0, M // L, combine, 0)
            pltpu.sync_copy(bx0, out.at[pl.ds(off, M)])

        prep(set_a, sem_a, 0)

        def pair(p, carry):
            s0 = 2 * p
            prep(set_b, sem_b, s0 + 1)
            finish(set_a, sem_a, s0)

            @pl.when(s0 + 2 < SUB)
            def _():
                prep(set_a, sem_a, s0 + 2)

            finish(set_b, sem_b, s0 + 1)
            return carry

        lax.fori_loop(0, SUB // 2, pair, 0)

    return triline


def kernel(coords, x_line, y_line, z_line, grid):
    B = coords.shape[0]
    N, C = x_line.shape
    xs = coords[:, 0]
    ys = coords[:, 1]
    zs = coords[:, 2]
    fn = _make_triline(B, N, C)
    return fn(xs, ys, zs, x_line, y_line, z_line, grid)
